# baseline (speedup 1.0000x reference)
import functools

import jax
import jax.numpy as jnp
from jax import lax
from jax.experimental import pallas as pl
from jax.experimental.pallas import tpu as pltpu
from jax.experimental.pallas import tpu_sc as plsc

_E = 64
_K = 2
_D = 1024
_I = 512
_IS = 1024
_BT = 64
_TMAX = 128
_NPAD = _TMAX * _BT
_NC, _NS = 2, 16
_NW = _NC * _NS


def _router_body(x_ref, gw_ref, w_ref, p_ref, te_ref, tv_ref):
    s = x_ref.shape[0]
    scores = jax.nn.sigmoid(
        jnp.dot(x_ref[...], gw_ref[...].T, preferred_element_type=jnp.float32))
    iota_e = lax.broadcasted_iota(jnp.int32, (s, _E), 1)
    m1 = jnp.max(scores, axis=1, keepdims=True)
    i1 = jnp.min(jnp.where(scores == m1, iota_e, _E), axis=1, keepdims=True)
    masked = jnp.where(iota_e == i1, -jnp.inf, scores)
    m2 = jnp.max(masked, axis=1, keepdims=True)
    i2 = jnp.min(jnp.where(masked == m2, iota_e, _E), axis=1, keepdims=True)
    w_ref[...] = jnp.concatenate([m1, m2], axis=1)

    oh = ((iota_e == i1) | (iota_e == i2)).astype(jnp.float32)
    r = lax.broadcasted_iota(jnp.int32, (s, s), 0)
    c = lax.broadcasted_iota(jnp.int32, (s, s), 1)
    ltri = (r > c).astype(jnp.float32)
    rank = jnp.dot(ltri, oh, preferred_element_type=jnp.float32)
    counts = jnp.sum(oh, axis=0, keepdims=True)
    ntiles = jnp.floor((counts + (_BT - 1)) * (1.0 / _BT))
    er = lax.broadcasted_iota(jnp.int32, (_E, _E), 0)
    ec = lax.broadcasted_iota(jnp.int32, (_E, _E), 1)
    utri = (er <= ec).astype(jnp.float32)
    tile_end = jnp.dot(ntiles, utri, preferred_element_type=jnp.float32)
    pad_off = (tile_end - ntiles) * _BT

    sel1 = (iota_e == i1).astype(jnp.float32)
    sel2 = (iota_e == i2).astype(jnp.float32)
    p1 = jnp.sum(sel1 * (pad_off + rank), axis=1, keepdims=True)
    p2 = jnp.sum(sel2 * (pad_off + rank), axis=1, keepdims=True)
    p_ref[...] = jnp.concatenate([p1, p2], axis=1).astype(jnp.int32)

    tq = lax.broadcasted_iota(jnp.int32, (_TMAX, 1), 0).astype(jnp.float32)
    te_raw = jnp.sum((tile_end <= tq).astype(jnp.int32), axis=1,
                     keepdims=True)
    total = jnp.sum(jnp.where(
        lax.broadcasted_iota(jnp.int32, (1, _E), 1) == _E - 1, tile_end,
        0.0), axis=1, keepdims=True)
    valid = tq < total
    last_e = jnp.sum(jnp.where(tq == total - 1.0, te_raw, 0), axis=0,
                     keepdims=True)
    te = jnp.where(valid, jnp.minimum(te_raw, _E - 1), last_e)
    te_ref[...] = te.astype(jnp.int32)
    tv_ref[...] = valid.astype(jnp.int32)


def _router(x, gate_w):
    s = x.shape[0]
    return pl.pallas_call(
        _router_body,
        out_shape=[jax.ShapeDtypeStruct((s, _K), jnp.float32),
                   jax.ShapeDtypeStruct((s, _K), jnp.int32),
                   jax.ShapeDtypeStruct((_TMAX, 1), jnp.int32),
                   jax.ShapeDtypeStruct((_TMAX, 1), jnp.int32)],
    )(x, gate_w)


@functools.lru_cache(maxsize=None)
def _sc_dispatch_fn(n_idx, n_rows, d, chunk):
    per_w = n_idx // _NW
    n_chunks = per_w // chunk
    mesh = plsc.VectorSubcoreMesh(
        core_axis_name="c", subcore_axis_name="s",
        num_cores=_NC, num_subcores=_NS)

    @functools.partial(
        pl.kernel,
        out_type=jax.ShapeDtypeStruct((_NPAD, d), jnp.float32),
        mesh=mesh,
        scratch_types=[
            pltpu.VMEM((chunk,), jnp.int32),
            pltpu.VMEM((chunk,), jnp.int32),
            pltpu.VMEM((chunk, d), jnp.float32),
            pltpu.SemaphoreType.DMA,
            pltpu.SemaphoreType.DMA,
        ],
    )
    def k(table_hbm, tok_hbm, ppos_hbm, out_hbm, tok_v, ppos_v, rows_v,
          sem_g, sem_s):
        wid = lax.axis_index("c") * _NS + lax.axis_index("s")
        base = wid * per_w
        for c in range(n_chunks):
            off = base + c * chunk
            pltpu.sync_copy(tok_hbm.at[pl.ds(off, chunk)], tok_v)
            pltpu.sync_copy(ppos_hbm.at[pl.ds(off, chunk)], ppos_v)
            pltpu.async_copy(table_hbm.at[tok_v], rows_v, sem_g).wait()
            pltpu.async_copy(rows_v, out_hbm.at[ppos_v], sem_s).wait()

    return k


def _sc_dispatch(table, ppos_flat, chunk=64):
    tok = jnp.arange(ppos_flat.shape[0], dtype=jnp.int32) // _K
    return _sc_dispatch_fn(ppos_flat.shape[0], table.shape[0],
                           table.shape[1], chunk)(table, tok, ppos_flat)


@functools.lru_cache(maxsize=None)
def _sc_gather_fn(n_idx, n_rows, d, chunk):
    per_w = n_idx // _NW
    n_chunks = per_w // chunk
    mesh = plsc.VectorSubcoreMesh(
        core_axis_name="c", subcore_axis_name="s",
        num_cores=_NC, num_subcores=_NS)

    @functools.partial(
        pl.kernel,
        out_type=jax.ShapeDtypeStruct((n_idx, d), jnp.float32),
        mesh=mesh,
        scratch_types=[
            pltpu.VMEM((chunk,), jnp.int32),
            pltpu.VMEM((chunk, d), jnp.float32),
            pltpu.SemaphoreType.DMA,
        ],
    )
    def k(table_hbm, idx_hbm, out_hbm, idx_v, rows_v, sem):
        wid = lax.axis_index("c") * _NS + lax.axis_index("s")
        base = wid * per_w
        for c in range(n_chunks):
            off = base + c * chunk
            pltpu.sync_copy(idx_hbm.at[pl.ds(off, chunk)], idx_v)
            pltpu.async_copy(table_hbm.at[idx_v], rows_v, sem).wait()
            pltpu.sync_copy(rows_v, out_hbm.at[pl.ds(off, chunk)])

    return k


def _sc_gather(table, idx, chunk=64):
    return _sc_gather_fn(idx.shape[0], table.shape[0], table.shape[1],
                         chunk)(table, idx)


def _expert_body(te_ref, tv_ref, xs_ref, w1g_ref, w1u_ref, w2a_ref, w2b_ref,
                 out_ref):
    t = pl.program_id(0)

    @pl.when(tv_ref[t] > 0)
    def _():
        xt = xs_ref[...]
        g = jnp.dot(xt, w1g_ref[0, 0].T, preferred_element_type=jnp.float32)
        u = jnp.dot(xt, w1u_ref[0, 0].T, preferred_element_type=jnp.float32)
        act = g * jax.nn.sigmoid(g) * u
        ya = jnp.dot(act, w2a_ref[0, 0].T, preferred_element_type=jnp.float32)
        yb = jnp.dot(act, w2b_ref[0, 0].T, preferred_element_type=jnp.float32)
        out_ref[...] = jnp.concatenate([ya, yb], axis=1)


def _expert_mlp(tile_expert, tile_valid, xs, w1, w2):
    w14 = w1.reshape(_E, 2, _I, _D)
    w24 = w2.reshape(_E, 2, _D // 2, _I)
    grid_spec = pltpu.PrefetchScalarGridSpec(
        num_scalar_prefetch=2,
        grid=(_TMAX,),
        in_specs=[
            pl.BlockSpec((_BT, _D),
                         lambda t, te, tv: (jnp.where(tv[t] > 0, t,
                                                      _TMAX - 1), 0)),
            pl.BlockSpec((1, 1, _I, _D), lambda t, te, tv: (te[t], 0, 0, 0)),
            pl.BlockSpec((1, 1, _I, _D), lambda t, te, tv: (te[t], 1, 0, 0)),
            pl.BlockSpec((1, 1, _D // 2, _I),
                         lambda t, te, tv: (te[t], 0, 0, 0)),
            pl.BlockSpec((1, 1, _D // 2, _I),
                         lambda t, te, tv: (te[t], 1, 0, 0)),
        ],
        out_specs=pl.BlockSpec((_BT, _D),
                               lambda t, te, tv: (jnp.where(tv[t] > 0, t,
                                                            _TMAX - 1), 0)),
    )
    return pl.pallas_call(
        _expert_body,
        grid_spec=grid_spec,
        out_shape=jax.ShapeDtypeStruct((_NPAD, _D), jnp.float32),
        compiler_params=pltpu.CompilerParams(
            dimension_semantics=("arbitrary",)),
    )(tile_expert, tile_valid, xs, w14, w14, w24, w24)


def _shared_body(x_ref, sgu_ref, sd_ref, g0_ref, g1_ref, tw_ref, o_ref):
    h = jnp.dot(x_ref[...], sgu_ref[...].T, preferred_element_type=jnp.float32)
    g = h[:, :_IS]
    u = h[:, _IS:]
    act = g * jax.nn.sigmoid(g) * u
    shared = jnp.dot(act, sd_ref[...].T, preferred_element_type=jnp.float32)
    moe = tw_ref[:, 0:1] * g0_ref[...] + tw_ref[:, 1:2] * g1_ref[...]
    o_ref[...] = (shared + _K * moe) / (_K + 1.0)


def _shared_combine(x, sgu, sd, g, topk_w):
    s = x.shape[0]
    sb = 256
    nb = s // sb
    return pl.pallas_call(
        _shared_body,
        grid=(nb,),
        in_specs=[
            pl.BlockSpec((sb, _D), lambda i: (i, 0)),
            pl.BlockSpec((2 * _IS, _D), lambda i: (0, 0)),
            pl.BlockSpec((_D, _IS), lambda i: (0, 0)),
            pl.BlockSpec((sb, _D), lambda i: (i, 0)),
            pl.BlockSpec((sb, _D), lambda i, _nb=nb: (i + _nb, 0)),
            pl.BlockSpec((sb, _K), lambda i: (i, 0)),
        ],
        out_specs=pl.BlockSpec((sb, _D), lambda i: (i, 0)),
        out_shape=jax.ShapeDtypeStruct((s, _D), jnp.float32),
    )(x, sgu, sd, g, g, topk_w)


def _impl(hidden_states, gate_w, w1, w2, shared_gate_up, shared_down):
    orig_shape = hidden_states.shape
    x = hidden_states.reshape(-1, orig_shape[-1])
    s = x.shape[0]

    topk_w, ppos, tile_expert, tile_valid = _router(x, gate_w)
    ppos_flat = ppos.reshape(-1)
    te = tile_expert.reshape(-1)
    tv = tile_valid.reshape(-1)

    xs = _sc_dispatch(x, ppos_flat)
    expanded = _expert_mlp(te, tv, xs, w1, w2)
    pq = jnp.concatenate([ppos[:, 0], ppos[:, 1]])
    g = _sc_gather(expanded, pq)
    final = _shared_combine(x, shared_gate_up, shared_down, g, topk_w)
    return final.reshape(orig_shape)


def kernel(hidden_states, gate_w, w1, w2, shared_gate_up, shared_down):
    return _impl(hidden_states, gate_w, w1, w2, shared_gate_up, shared_down)


# speedup vs baseline: 1.0128x; 1.0128x over previous
import functools

import jax
import jax.numpy as jnp
from jax import lax
from jax.experimental import pallas as pl
from jax.experimental.pallas import tpu as pltpu
from jax.experimental.pallas import tpu_sc as plsc

_E = 64
_K = 2
_D = 1024
_I = 512
_IS = 1024
_BT = 64
_TMAX = 128
_NPAD = _TMAX * _BT
_NC, _NS = 2, 16
_NW = _NC * _NS


def _router_body(x_ref, gw_ref, w_ref, p_ref, te_ref, tv_ref):
    s = x_ref.shape[0]
    scores = jax.nn.sigmoid(
        jnp.dot(x_ref[...], gw_ref[...].T, preferred_element_type=jnp.float32))
    iota_e = lax.broadcasted_iota(jnp.int32, (s, _E), 1)
    m1 = jnp.max(scores, axis=1, keepdims=True)
    i1 = jnp.min(jnp.where(scores == m1, iota_e, _E), axis=1, keepdims=True)
    masked = jnp.where(iota_e == i1, -jnp.inf, scores)
    m2 = jnp.max(masked, axis=1, keepdims=True)
    i2 = jnp.min(jnp.where(masked == m2, iota_e, _E), axis=1, keepdims=True)
    w_ref[...] = jnp.concatenate([m1, m2], axis=1)

    oh = ((iota_e == i1) | (iota_e == i2)).astype(jnp.float32)
    cb = 512
    r = lax.broadcasted_iota(jnp.int32, (cb, cb), 0)
    c = lax.broadcasted_iota(jnp.int32, (cb, cb), 1)
    ltri = (r > c).astype(jnp.float32)
    parts = []
    carry = jnp.zeros((1, _E), jnp.float32)
    for b in range(s // cb):
        ohb = oh[b * cb:(b + 1) * cb, :]
        parts.append(jnp.dot(ltri, ohb, preferred_element_type=jnp.float32)
                     + carry)
        carry = carry + jnp.sum(ohb, axis=0, keepdims=True)
    rank = jnp.concatenate(parts, axis=0)
    counts = carry
    ntiles = jnp.floor((counts + (_BT - 1)) * (1.0 / _BT))
    er = lax.broadcasted_iota(jnp.int32, (_E, _E), 0)
    ec = lax.broadcasted_iota(jnp.int32, (_E, _E), 1)
    utri = (er <= ec).astype(jnp.float32)
    tile_end = jnp.dot(ntiles, utri, preferred_element_type=jnp.float32)
    pad_off = (tile_end - ntiles) * _BT

    sel1 = (iota_e == i1).astype(jnp.float32)
    sel2 = (iota_e == i2).astype(jnp.float32)
    p1 = jnp.sum(sel1 * (pad_off + rank), axis=1, keepdims=True)
    p2 = jnp.sum(sel2 * (pad_off + rank), axis=1, keepdims=True)
    p_ref[...] = jnp.concatenate([p1, p2], axis=1).astype(jnp.int32)

    tq = lax.broadcasted_iota(jnp.int32, (_TMAX, 1), 0).astype(jnp.float32)
    te_raw = jnp.sum((tile_end <= tq).astype(jnp.int32), axis=1,
                     keepdims=True)
    total = jnp.sum(jnp.where(
        lax.broadcasted_iota(jnp.int32, (1, _E), 1) == _E - 1, tile_end,
        0.0), axis=1, keepdims=True)
    valid = tq < total
    last_e = jnp.sum(jnp.where(tq == total - 1.0, te_raw, 0), axis=0,
                     keepdims=True)
    te = jnp.where(valid, jnp.minimum(te_raw, _E - 1), last_e)
    te_ref[...] = te.astype(jnp.int32)
    tv_ref[...] = valid.astype(jnp.int32)


def _router(x, gate_w):
    s = x.shape[0]
    return pl.pallas_call(
        _router_body,
        out_shape=[jax.ShapeDtypeStruct((s, _K), jnp.float32),
                   jax.ShapeDtypeStruct((s, _K), jnp.int32),
                   jax.ShapeDtypeStruct((_TMAX, 1), jnp.int32),
                   jax.ShapeDtypeStruct((_TMAX, 1), jnp.int32)],
    )(x, gate_w)


@functools.lru_cache(maxsize=None)
def _sc_dispatch_fn(n_idx, n_rows, d, chunk):
    per_w = n_idx // _NW
    n_chunks = per_w // chunk
    mesh = plsc.VectorSubcoreMesh(
        core_axis_name="c", subcore_axis_name="s",
        num_cores=_NC, num_subcores=_NS)

    @functools.partial(
        pl.kernel,
        out_type=jax.ShapeDtypeStruct((_NPAD, d), jnp.float32),
        mesh=mesh,
        scratch_types=[
            pltpu.VMEM((chunk,), jnp.int32),
            pltpu.VMEM((chunk,), jnp.int32),
            pltpu.VMEM((chunk, d), jnp.float32),
            pltpu.SemaphoreType.DMA,
            pltpu.SemaphoreType.DMA,
        ],
    )
    def k(table_hbm, tok_hbm, ppos_hbm, out_hbm, tok_v, ppos_v, rows_v,
          sem_g, sem_s):
        wid = lax.axis_index("c") * _NS + lax.axis_index("s")
        base = wid * per_w
        for c in range(n_chunks):
            off = base + c * chunk
            pltpu.sync_copy(tok_hbm.at[pl.ds(off, chunk)], tok_v)
            pltpu.sync_copy(ppos_hbm.at[pl.ds(off, chunk)], ppos_v)
            pltpu.async_copy(table_hbm.at[tok_v], rows_v, sem_g).wait()
            pltpu.async_copy(rows_v, out_hbm.at[ppos_v], sem_s).wait()

    return k


def _sc_dispatch(table, ppos_flat, chunk=64):
    tok = jnp.arange(ppos_flat.shape[0], dtype=jnp.int32) // _K
    return _sc_dispatch_fn(ppos_flat.shape[0], table.shape[0],
                           table.shape[1], chunk)(table, tok, ppos_flat)


@functools.lru_cache(maxsize=None)
def _sc_gather_fn(n_idx, n_rows, d, chunk):
    per_w = n_idx // _NW
    n_chunks = per_w // chunk
    mesh = plsc.VectorSubcoreMesh(
        core_axis_name="c", subcore_axis_name="s",
        num_cores=_NC, num_subcores=_NS)

    @functools.partial(
        pl.kernel,
        out_type=jax.ShapeDtypeStruct((n_idx, d), jnp.float32),
        mesh=mesh,
        scratch_types=[
            pltpu.VMEM((chunk,), jnp.int32),
            pltpu.VMEM((chunk, d), jnp.float32),
            pltpu.SemaphoreType.DMA,
        ],
    )
    def k(table_hbm, idx_hbm, out_hbm, idx_v, rows_v, sem):
        wid = lax.axis_index("c") * _NS + lax.axis_index("s")
        base = wid * per_w
        for c in range(n_chunks):
            off = base + c * chunk
            pltpu.sync_copy(idx_hbm.at[pl.ds(off, chunk)], idx_v)
            pltpu.async_copy(table_hbm.at[idx_v], rows_v, sem).wait()
            pltpu.sync_copy(rows_v, out_hbm.at[pl.ds(off, chunk)])

    return k


def _sc_gather(table, idx, chunk=64):
    return _sc_gather_fn(idx.shape[0], table.shape[0], table.shape[1],
                         chunk)(table, idx)


def _expert_body(te_ref, tv_ref, xs_ref, w1_ref, w2_ref, out_ref):
    t = pl.program_id(0)

    @pl.when(tv_ref[t] > 0)
    def _():
        h = jnp.dot(xs_ref[...], w1_ref[0].T,
                    preferred_element_type=jnp.float32)
        g = h[:, :_I]
        u = h[:, _I:]
        act = g * jax.nn.sigmoid(g) * u
        out_ref[...] = jnp.dot(act, w2_ref[0].T,
                               preferred_element_type=jnp.float32)


def _expert_mlp(tile_expert, tile_valid, xs, w1, w2):
    grid_spec = pltpu.PrefetchScalarGridSpec(
        num_scalar_prefetch=2,
        grid=(_TMAX,),
        in_specs=[
            pl.BlockSpec((_BT, _D),
                         lambda t, te, tv: (jnp.where(tv[t] > 0, t,
                                                      _TMAX - 1), 0)),
            pl.BlockSpec((1, 2 * _I, _D), lambda t, te, tv: (te[t], 0, 0)),
            pl.BlockSpec((1, _D, _I), lambda t, te, tv: (te[t], 0, 0)),
        ],
        out_specs=pl.BlockSpec((_BT, _D),
                               lambda t, te, tv: (jnp.where(tv[t] > 0, t,
                                                            _TMAX - 1), 0)),
    )
    return pl.pallas_call(
        _expert_body,
        grid_spec=grid_spec,
        out_shape=jax.ShapeDtypeStruct((_NPAD, _D), jnp.float32),
        compiler_params=pltpu.CompilerParams(
            dimension_semantics=("arbitrary",)),
    )(tile_expert, tile_valid, xs, w1, w2)


def _shared_body(x_ref, sgu_ref, sd_ref, o_ref):
    h = jnp.dot(x_ref[...], sgu_ref[...].T, preferred_element_type=jnp.float32)
    g = h[:, :_IS]
    u = h[:, _IS:]
    act = g * jax.nn.sigmoid(g) * u
    o_ref[...] = jnp.dot(act, sd_ref[...].T, preferred_element_type=jnp.float32)


def _shared_mlp(x, sgu, sd):
    s = x.shape[0]
    sb = 256
    return pl.pallas_call(
        _shared_body,
        grid=(s // sb,),
        in_specs=[
            pl.BlockSpec((sb, _D), lambda i: (i, 0)),
            pl.BlockSpec((2 * _IS, _D), lambda i: (0, 0)),
            pl.BlockSpec((_D, _IS), lambda i: (0, 0)),
        ],
        out_specs=pl.BlockSpec((sb, _D), lambda i: (i, 0)),
        out_shape=jax.ShapeDtypeStruct((s, _D), jnp.float32),
    )(x, sgu, sd)


def _combine_body(sh_ref, g0_ref, g1_ref, tw_ref, o_ref):
    moe = tw_ref[:, 0:1] * g0_ref[...] + tw_ref[:, 1:2] * g1_ref[...]
    o_ref[...] = (sh_ref[...] + _K * moe) / (_K + 1.0)


def _combine(shared, g, topk_w):
    s = shared.shape[0]
    sb = 256
    nb = s // sb
    return pl.pallas_call(
        _combine_body,
        grid=(nb,),
        in_specs=[
            pl.BlockSpec((sb, _D), lambda i: (i, 0)),
            pl.BlockSpec((sb, _D), lambda i: (i, 0)),
            pl.BlockSpec((sb, _D), lambda i, _nb=nb: (i + _nb, 0)),
            pl.BlockSpec((sb, _K), lambda i: (i, 0)),
        ],
        out_specs=pl.BlockSpec((sb, _D), lambda i: (i, 0)),
        out_shape=jax.ShapeDtypeStruct((s, _D), jnp.float32),
    )(shared, g, g, topk_w)


def _impl(hidden_states, gate_w, w1, w2, shared_gate_up, shared_down):
    orig_shape = hidden_states.shape
    x = hidden_states.reshape(-1, orig_shape[-1])
    s = x.shape[0]

    shared = _shared_mlp(x, shared_gate_up, shared_down)
    topk_w, ppos, tile_expert, tile_valid = _router(x, gate_w)
    ppos_flat = ppos.reshape(-1)
    te = tile_expert.reshape(-1)
    tv = tile_valid.reshape(-1)

    xs = _sc_dispatch(x, ppos_flat)
    expanded = _expert_mlp(te, tv, xs, w1, w2)
    pq = jnp.concatenate([ppos[:, 0], ppos[:, 1]])
    g = _sc_gather(expanded, pq)
    final = _combine(shared, g, topk_w)
    return final.reshape(orig_shape)


def kernel(hidden_states, gate_w, w1, w2, shared_gate_up, shared_down):
    return _impl(hidden_states, gate_w, w1, w2, shared_gate_up, shared_down)
